# baseline (speedup 1.0000x reference)
import functools

import jax
import jax.numpy as jnp
import numpy as np
from jax import lax
from jax.experimental import pallas as pl
from jax.experimental.pallas import tpu as pltpu
from jax.experimental.pallas import tpu_sc as plsc

_NUM_FEATURES = 26
_ROWS_PER_FEATURE = 100000
_CHUNK = 128
_SEG = 1024


@functools.cache
def _build_transpose(dim, rows, nw):
    nseg = (rows // _SEG)
    rows_main = nseg * _SEG
    tail = rows - rows_main
    n_iter = (nseg + nw - 1) // nw
    tps = _SEG // 128
    out_rows = rows * dim // 128
    tail_out = tail * dim // 128
    mesh = plsc.VectorSubcoreMesh(core_axis_name="c", subcore_axis_name="s")

    @functools.partial(
        pl.kernel,
        out_type=jax.ShapeDtypeStruct((out_rows, 128), jnp.float32),
        mesh=mesh,
        compiler_params=pltpu.CompilerParams(
            use_tc_tiling_on_sc=True, needs_layout_passes=False
        ),
        scratch_types=[
            pltpu.VMEM((2, 2 * tps, 8, 128), jnp.float32),
            pltpu.VMEM((2, _SEG * dim // 128, 128), jnp.float32),
            pltpu.SemaphoreType.DMA,
            pltpu.SemaphoreType.DMA,
            pltpu.SemaphoreType.DMA,
        ],
    )
    def k(src_hbm, tail_hbm, dst_hbm, in_v, out_v, sem0, sem1, osem):
        wid = lax.axis_index("s") * 2 + lax.axis_index("c")
        lanes = lax.iota(jnp.int32, 16)
        rowc = lanes >> 3
        colc = (lanes & 7) * dim

        def fire(s, buf, sem):
            c0 = s * _SEG
            for h in range(2):
                for j in range(tps):
                    pltpu.async_copy(
                        src_hbm.at[h, :, pl.ds(c0 + j * 128, 128)],
                        in_v.at[buf, h * tps + j],
                        sem,
                    )

        def drain(s, buf, sem):
            c0 = s * _SEG
            for h in range(2):
                for j in range(tps):
                    pltpu.make_async_copy(
                        src_hbm.at[h, :, pl.ds(c0 + j * 128, 128)],
                        in_v.at[buf, h * tps + j],
                        sem,
                    ).wait()

        oseg = _SEG * dim // 128

        def process(s, buf, sem):
            drain(s, buf, sem)

            def t_body(t, c2):
                h = t >> 3
                j = t & 7
                jb = j * 16
                for d in range(8):
                    col_idx = colc + (h * 8 + d)
                    for lb in range(8):
                        v = in_v[buf, t, d, pl.ds(lb * 16, 16)]
                        row_idx = rowc + (jb + lb * 2)
                        plsc.store_scatter(out_v.at[buf], [row_idx, col_idx], v)
                return c2

            lax.fori_loop(0, 2 * tps, t_body, 0, unroll=2)
            pltpu.async_copy(
                out_v.at[buf], dst_hbm.at[pl.ds(s * oseg, oseg)], osem
            )

        def owait(s, buf):
            pltpu.make_async_copy(
                out_v.at[buf], dst_hbm.at[pl.ds(s * oseg, oseg)], osem
            ).wait()

        @pl.when(wid < nseg)
        def _():
            fire(wid, 0, sem0)

        def seg_body(i, carry):
            sa = (2 * i) * nw + wid
            sb = sa + nw
            sc = sa + 2 * nw

            @pl.when(sb < nseg)
            def _():
                fire(sb, 1, sem1)

            @pl.when(sa < nseg)
            def _():
                process(sa, 0, sem0)

            @pl.when(sc < nseg)
            def _():
                fire(sc, 0, sem0)

            @pl.when(sb < nseg)
            def _():
                process(sb, 1, sem1)

            @pl.when(sa < nseg)
            def _():
                owait(sa, 0)

            @pl.when(sb < nseg)
            def _():
                owait(sb, 1)

            return carry

        lax.fori_loop(0, (n_iter + 1) // 2, seg_body, 0)

        if tail:
            @pl.when(wid == nw - 1)
            def _():
                pltpu.sync_copy(tail_hbm, out_v.at[0, pl.ds(0, tail_out)])
                pltpu.sync_copy(
                    out_v.at[0, pl.ds(0, tail_out)],
                    dst_hbm.at[pl.ds(out_rows - tail_out, tail_out)],
                )

    return k


@functools.cache
def _build_lookup(batch, feat, dim, rows, nw):
    rows_per_w = batch // nw
    n_chunks = rows_per_w // _CHUNK
    chf = _CHUNK * feat
    mesh = plsc.VectorSubcoreMesh(core_axis_name="c", subcore_axis_name="s")

    @functools.partial(
        pl.kernel,
        out_type=jax.ShapeDtypeStruct((batch, dim), jnp.float32),
        mesh=mesh,
        compiler_params=pltpu.CompilerParams(use_tc_tiling_on_sc=False),
        scratch_types=[
            pltpu.VMEM((chf,), jnp.int32),
            pltpu.VMEM((chf,), jnp.int32),
            pltpu.VMEM((chf, dim), jnp.float32),
            pltpu.VMEM((_CHUNK, dim), jnp.float32),
            pltpu.SemaphoreType.DMA,
        ],
    )
    def k(x_hbm, offs_hbm, table_hbm, out_hbm, idx_v, offs_v, rows_v, acc_v, sem):
        wid = lax.axis_index("s") * 2 + lax.axis_index("c")
        base = wid * rows_per_w
        pltpu.sync_copy(offs_hbm, offs_v)

        def chunk_body(c, carry):
            cb = base + c * _CHUNK
            pltpu.sync_copy(x_hbm.at[pl.ds(cb * feat, chf)], idx_v)

            def add_body(i, carry2):
                s = i * 16
                idx_v[pl.ds(s, 16)] = idx_v[pl.ds(s, 16)] + offs_v[pl.ds(s, 16)]
                return carry2

            lax.fori_loop(0, chf // 16, add_body, 0, unroll=8)

            pltpu.async_copy(table_hbm.at[idx_v], rows_v, sem).wait()

            def sum_rows(b, carry3):
                a = rows_v.at[b * feat][...]
                for j in range(1, feat):
                    a = a + rows_v.at[b * feat + j][...]
                acc_v.at[b][...] = a
                return carry3

            lax.fori_loop(0, _CHUNK, sum_rows, 0)
            pltpu.sync_copy(acc_v, out_hbm.at[pl.ds(cb, _CHUNK)])
            return carry

        lax.fori_loop(0, n_chunks, chunk_body, 0)

    return k


def kernel(x, table):
    batch, feat = x.shape
    rows, dim = table.shape
    info = plsc.get_sparse_core_info()
    nw = info.num_cores * info.num_subcores
    rows_main = (rows // _SEG) * _SEG
    offsets = np.arange(feat, dtype=np.int32) * _ROWS_PER_FEATURE
    offs_rep = jnp.asarray(np.tile(offsets, _CHUNK))
    x_flat = x.reshape(-1).astype(jnp.int32)
    tableT3 = table.T.reshape(2, dim // 2, rows)
    tail_packed = table[rows_main:].reshape(-1, 128)
    packed = _build_transpose(dim, rows, nw)(tableT3, tail_packed)
    table_rm = packed.reshape(rows, dim)
    return _build_lookup(batch, feat, dim, rows, nw)(x_flat, offs_rep, table_rm)


# speedup vs baseline: 1.0256x; 1.0256x over previous
import functools

import jax
import jax.numpy as jnp
import numpy as np
from jax import lax
from jax.experimental import pallas as pl
from jax.experimental.pallas import tpu as pltpu
from jax.experimental.pallas import tpu_sc as plsc

_NUM_FEATURES = 26
_ROWS_PER_FEATURE = 100000
_CHUNK = 128
_SEG = 1024


@functools.cache
def _build_transpose(dim, rows, nw):
    nseg = (rows // _SEG)
    rows_main = nseg * _SEG
    tail = rows - rows_main
    n_iter = (nseg + nw - 1) // nw
    tps = _SEG // 128
    out_rows = rows * dim // 128
    tail_out = tail * dim // 128
    mesh = plsc.VectorSubcoreMesh(core_axis_name="c", subcore_axis_name="s")

    @functools.partial(
        pl.kernel,
        out_type=jax.ShapeDtypeStruct((out_rows, 128), jnp.float32),
        mesh=mesh,
        compiler_params=pltpu.CompilerParams(
            use_tc_tiling_on_sc=True, needs_layout_passes=False
        ),
        scratch_types=[
            pltpu.VMEM((2, 2 * tps, 8, 128), jnp.float32),
            pltpu.VMEM((2, _SEG * dim // 128, 128), jnp.float32),
            pltpu.SemaphoreType.DMA,
            pltpu.SemaphoreType.DMA,
            pltpu.SemaphoreType.DMA,
        ],
    )
    def k(src_hbm, tail_hbm, dst_hbm, in_v, out_v, sem0, sem1, osem):
        wid = lax.axis_index("s") * 2 + lax.axis_index("c")
        lanes = lax.iota(jnp.int32, 16)
        rowc = lanes >> 3
        colc = (lanes & 7) * dim

        def fire(s, buf, sem):
            c0 = s * _SEG
            for h in range(2):
                for j in range(tps):
                    pltpu.async_copy(
                        src_hbm.at[h, :, pl.ds(c0 + j * 128, 128)],
                        in_v.at[buf, h * tps + j],
                        sem,
                    )

        def drain(s, buf, sem):
            c0 = s * _SEG
            for h in range(2):
                for j in range(tps):
                    pltpu.make_async_copy(
                        src_hbm.at[h, :, pl.ds(c0 + j * 128, 128)],
                        in_v.at[buf, h * tps + j],
                        sem,
                    ).wait()

        oseg = _SEG * dim // 128

        def process(s, buf, sem):
            drain(s, buf, sem)

            def t_body(t, c2):
                h = t >> 3
                j = t & 7
                jb = j * 16
                for d in range(8):
                    col_idx = colc + (h * 8 + d)
                    for lb in range(8):
                        v = in_v[buf, t, d, pl.ds(lb * 16, 16)]
                        row_idx = rowc + (jb + lb * 2)
                        plsc.store_scatter(out_v.at[buf], [row_idx, col_idx], v)
                return c2

            lax.fori_loop(0, 2 * tps, t_body, 0)
            pltpu.async_copy(
                out_v.at[buf], dst_hbm.at[pl.ds(s * oseg, oseg)], osem
            )

        def owait(s, buf):
            pltpu.make_async_copy(
                out_v.at[buf], dst_hbm.at[pl.ds(s * oseg, oseg)], osem
            ).wait()

        @pl.when(wid < nseg)
        def _():
            fire(wid, 0, sem0)

        def seg_body(i, carry):
            sa = (2 * i) * nw + wid
            sb = sa + nw
            sc = sa + 2 * nw

            @pl.when(sb < nseg)
            def _():
                fire(sb, 1, sem1)

            @pl.when(sa < nseg)
            def _():
                process(sa, 0, sem0)

            @pl.when(sc < nseg)
            def _():
                fire(sc, 0, sem0)

            @pl.when(sb < nseg)
            def _():
                process(sb, 1, sem1)

            @pl.when(sa < nseg)
            def _():
                owait(sa, 0)

            @pl.when(sb < nseg)
            def _():
                owait(sb, 1)

            return carry

        lax.fori_loop(0, (n_iter + 1) // 2, seg_body, 0)

        if tail:
            @pl.when(wid == nw - 1)
            def _():
                pltpu.sync_copy(tail_hbm, out_v.at[0, pl.ds(0, tail_out)])
                pltpu.sync_copy(
                    out_v.at[0, pl.ds(0, tail_out)],
                    dst_hbm.at[pl.ds(out_rows - tail_out, tail_out)],
                )

    return k


@functools.cache
def _build_lookup(batch, feat, dim, rows, nw):
    rows_per_w = batch // nw
    n_chunks = rows_per_w // _CHUNK
    chf = _CHUNK * feat
    mesh = plsc.VectorSubcoreMesh(core_axis_name="c", subcore_axis_name="s")

    @functools.partial(
        pl.kernel,
        out_type=jax.ShapeDtypeStruct((batch, dim), jnp.float32),
        mesh=mesh,
        compiler_params=pltpu.CompilerParams(use_tc_tiling_on_sc=False),
        scratch_types=[
            pltpu.VMEM((2, chf), jnp.int32),
            pltpu.VMEM((chf,), jnp.int32),
            pltpu.VMEM((2, chf, dim), jnp.float32),
            pltpu.VMEM((_CHUNK, dim), jnp.float32),
            pltpu.SemaphoreType.DMA,
            pltpu.SemaphoreType.DMA,
        ],
    )
    def k(x_hbm, offs_hbm, table_hbm, out_hbm, idx_v, offs_v, rows_v, acc_v,
          sem0, sem1):
        wid = lax.axis_index("s") * 2 + lax.axis_index("c")
        base = wid * rows_per_w
        pltpu.sync_copy(offs_hbm, offs_v)
        sems = (sem0, sem1)

        def fire(c):
            buf = c % 2
            cb = base + c * _CHUNK
            pltpu.sync_copy(x_hbm.at[pl.ds(cb * feat, chf)], idx_v.at[buf])

            def add_body(i, carry2):
                s = i * 16
                idx_v[buf, pl.ds(s, 16)] = (
                    idx_v[buf, pl.ds(s, 16)] + offs_v[pl.ds(s, 16)]
                )
                return carry2

            lax.fori_loop(0, chf // 16, add_body, 0, unroll=8)
            pltpu.async_copy(
                table_hbm.at[idx_v.at[buf]], rows_v.at[buf], sems[buf]
            )

        def finish(c):
            buf = c % 2
            cb = base + c * _CHUNK
            pltpu.make_async_copy(
                table_hbm.at[idx_v.at[buf]], rows_v.at[buf], sems[buf]
            ).wait()

            def sum_rows(b, carry3):
                a = rows_v.at[buf, b * feat][...]
                for j in range(1, feat):
                    a = a + rows_v.at[buf, b * feat + j][...]
                acc_v.at[b][...] = a
                return carry3

            lax.fori_loop(0, _CHUNK, sum_rows, 0)
            pltpu.sync_copy(acc_v, out_hbm.at[pl.ds(cb, _CHUNK)])

        fire(0)
        for c in range(n_chunks):
            if c + 1 < n_chunks:
                fire(c + 1)
            finish(c)

    return k


def kernel(x, table):
    batch, feat = x.shape
    rows, dim = table.shape
    info = plsc.get_sparse_core_info()
    nw = info.num_cores * info.num_subcores
    rows_main = (rows // _SEG) * _SEG
    offsets = np.arange(feat, dtype=np.int32) * _ROWS_PER_FEATURE
    offs_rep = jnp.asarray(np.tile(offsets, _CHUNK))
    x_flat = x.reshape(-1).astype(jnp.int32)
    tableT3 = table.T.reshape(2, dim // 2, rows)
    tail_packed = table[rows_main:].reshape(-1, 128)
    packed = _build_transpose(dim, rows, nw)(tableT3, tail_packed)
    table_rm = packed.reshape(rows, dim)
    return _build_lookup(batch, feat, dim, rows, nw)(x_flat, offs_rep, table_rm)
